# baseline (speedup 1.0000x reference)
import functools

import jax
import jax.numpy as jnp
from jax import lax
from jax.experimental import pallas as pl
from jax.experimental.pallas import tpu as pltpu
from jax.experimental.pallas import tpu_sc as plsc

NUM_EMBEDDINGS = 1000000
EMBED_DIM = 64
BATCH = 16384

_NC = 2
_NS = 16
_NW = _NC * _NS
_BPW = BATCH // _NW
_CHUNK = 256

_mesh = plsc.VectorSubcoreMesh(core_axis_name="c", subcore_axis_name="s")


@functools.partial(
    pl.kernel,
    mesh=_mesh,
    out_type=(
        jax.ShapeDtypeStruct((BATCH, 128), jnp.float32),
        jax.ShapeDtypeStruct((BATCH, 128), jnp.float32),
    ),
    scratch_types=[
        pltpu.VMEM((_CHUNK,), jnp.int32),
        pltpu.VMEM((_CHUNK, 128), jnp.float32),
        pltpu.VMEM((_CHUNK,), jnp.int32),
        pltpu.VMEM((_CHUNK, 128), jnp.float32),
        pltpu.SemaphoreType.DMA,
        pltpu.SemaphoreType.DMA,
    ],
    compiler_params=pltpu.CompilerParams(use_tc_tiling_on_sc=True),
)
def _gather2(ut2, it2, uidx, iidx, uout, iout,
             uidx_v, urows_v, iidx_v, irows_v, usem, isem):
    wid = lax.axis_index("s") * _NC + lax.axis_index("c")
    for half in range(_BPW // _CHUNK):
        base = wid * _BPW + half * _CHUNK
        pltpu.sync_copy(uidx.at[pl.ds(base, _CHUNK)], uidx_v)
        pltpu.sync_copy(iidx.at[pl.ds(base, _CHUNK)], iidx_v)
        cu = pltpu.async_copy(ut2.at[uidx_v], urows_v, usem)
        ci = pltpu.async_copy(it2.at[iidx_v], irows_v, isem)
        cu.wait()
        pltpu.sync_copy(urows_v, uout.at[pl.ds(base, _CHUNK)])
        ci.wait()
        pltpu.sync_copy(irows_v, iout.at[pl.ds(base, _CHUNK)])


def kernel(user_table, item_table, user_idx, item_idx):
    ut2 = jnp.reshape(user_table, (NUM_EMBEDDINGS // 2, 128))
    it2 = jnp.reshape(item_table, (NUM_EMBEDDINGS // 2, 128))
    uo, io = _gather2(ut2, it2, user_idx >> 1, item_idx >> 1)
    u_odd = (user_idx & 1)[:, None] > 0
    i_odd = (item_idx & 1)[:, None] > 0
    u = jnp.where(u_odd, uo[:, 64:], uo[:, :64])
    i = jnp.where(i_odd, io[:, 64:], io[:, :64])
    return (u, i)


# speedup vs baseline: 1.0109x; 1.0109x over previous
import functools

import jax
import jax.numpy as jnp
from jax import lax
from jax.experimental import pallas as pl
from jax.experimental.pallas import tpu as pltpu
from jax.experimental.pallas import tpu_sc as plsc

NUM_EMBEDDINGS = 1000000
EMBED_DIM = 64
BATCH = 16384

_NC = 2
_NS = 16
_NW = _NC * _NS
_BPW = BATCH // _NW

_mesh = plsc.VectorSubcoreMesh(core_axis_name="c", subcore_axis_name="s")


@functools.partial(
    pl.kernel,
    mesh=_mesh,
    out_type=(
        jax.ShapeDtypeStruct((BATCH, EMBED_DIM), jnp.float32),
        jax.ShapeDtypeStruct((BATCH, EMBED_DIM), jnp.float32),
    ),
    scratch_types=[
        pltpu.VMEM((_BPW,), jnp.int32),
        pltpu.VMEM((_BPW, EMBED_DIM), jnp.float32),
        pltpu.VMEM((_BPW,), jnp.int32),
        pltpu.VMEM((_BPW, EMBED_DIM), jnp.float32),
        pltpu.SemaphoreType.DMA,
        pltpu.SemaphoreType.DMA,
    ],
    compiler_params=pltpu.CompilerParams(use_tc_tiling_on_sc=False),
)
def _gather2(user_table, item_table, user_idx, item_idx,
             user_out, item_out,
             uidx_v, urows_v, iidx_v, irows_v, usem, isem):
    wid = lax.axis_index("s") * _NC + lax.axis_index("c")
    base = wid * _BPW
    pltpu.sync_copy(user_idx.at[pl.ds(base, _BPW)], uidx_v)
    pltpu.sync_copy(item_idx.at[pl.ds(base, _BPW)], iidx_v)
    cu = pltpu.async_copy(user_table.at[uidx_v], urows_v, usem)
    ci = pltpu.async_copy(item_table.at[iidx_v], irows_v, isem)
    cu.wait()
    pltpu.sync_copy(urows_v, user_out.at[pl.ds(base, _BPW)])
    ci.wait()
    pltpu.sync_copy(irows_v, item_out.at[pl.ds(base, _BPW)])


def kernel(user_table, item_table, user_idx, item_idx):
    return _gather2(user_table, item_table, user_idx, item_idx)
